# Initial kernel scaffold; baseline (speedup 1.0000x reference)
import jax
import jax.numpy as jnp
from jax.experimental import pallas as pl


def kernel(id_embedding, edge_index, W1, b1, W2, b2):
    raise NotImplementedError("write your pallas kernel here")


# speedup vs baseline: 7.8661x; 7.8661x over previous
import functools

import jax
import jax.numpy as jnp
from jax import lax
from jax.experimental import pallas as pl
from jax.experimental.pallas import tpu as pltpu
from jax.experimental.pallas import tpu_sc as plsc

NC = 2
NS = 16
NW = NC * NS
LANES = 128
DEGW = 16


def _pad_up(x, m):
    return (x + m - 1) // m * m


def _make_deg(npad, nrows, d):
    rpt = nrows // NW
    per_tile = npad // NS
    mesh = plsc.VectorSubcoreMesh(core_axis_name="c", subcore_axis_name="s")

    @functools.partial(
        pl.kernel,
        mesh=mesh,
        out_type=jax.ShapeDtypeStruct((NC, npad, d), jnp.float32),
        scratch_types=[
            pltpu.VMEM((LANES,), jnp.int32),
            pltpu.VMEM((LANES,), jnp.int32),
            pltpu.VMEM((LANES, d), jnp.float32),
            pltpu.VMEM_SHARED((npad, d), jnp.float32),
        ],
    )
    def deg_kernel(ones2d, srcp, dstp, out, idx_s, idx_d, ones_v, deg_sh):
        c = lax.axis_index("c")
        s = lax.axis_index("s")
        wid = s * NC + c
        base = s * per_tile
        pltpu.sync_copy(ones2d.at[pl.ds(base, per_tile)],
                        deg_sh.at[pl.ds(base, per_tile)])
        pltpu.sync_copy(ones2d.at[pl.ds(0, LANES)], ones_v)
        plsc.subcore_barrier()

        def step(j, carry):
            r = wid * rpt + j
            pltpu.sync_copy(srcp.at[r], idx_s)
            pltpu.sync_copy(dstp.at[r], idx_d)
            pltpu.sync_copy(ones_v, deg_sh.at[idx_d], add=True)
            pltpu.sync_copy(ones_v, deg_sh.at[idx_s], add=True)
            return carry

        lax.fori_loop(0, rpt, step, 0)
        plsc.subcore_barrier()
        pltpu.sync_copy(deg_sh.at[pl.ds(base, per_tile)],
                        out.at[c, pl.ds(base, per_tile)])

    return deg_kernel


def _make_agg(npad, nrows, d):
    rpt = nrows // NW
    per_tile = npad // NS
    mesh = plsc.VectorSubcoreMesh(core_axis_name="c", subcore_axis_name="s")

    @functools.partial(
        pl.kernel,
        mesh=mesh,
        out_type=jax.ShapeDtypeStruct((NC, npad, d), jnp.float32),
        scratch_types=[
            pltpu.VMEM((LANES,), jnp.int32),
            pltpu.VMEM((LANES,), jnp.int32),
            pltpu.VMEM((LANES, d), jnp.float32),
            pltpu.VMEM_SHARED((npad, d), jnp.float32),
            pltpu.SemaphoreType.DMA,
        ],
    )
    def agg_kernel(g, srcp, dstp, out, idx_s, idx_d, rows, acc_sh, sem):
        c = lax.axis_index("c")
        s = lax.axis_index("s")
        wid = s * NC + c
        base = s * per_tile
        pltpu.sync_copy(g.at[pl.ds(base, per_tile)],
                        acc_sh.at[pl.ds(base, per_tile)])
        plsc.subcore_barrier()

        def step(j, carry):
            r = wid * rpt + j
            pltpu.sync_copy(srcp.at[r], idx_s)
            pltpu.sync_copy(dstp.at[r], idx_d)
            pltpu.async_copy(g.at[idx_s], rows, sem).wait()
            pltpu.sync_copy(rows, acc_sh.at[idx_d], add=True)
            pltpu.async_copy(g.at[idx_d], rows, sem).wait()
            pltpu.sync_copy(rows, acc_sh.at[idx_s], add=True)
            return carry

        lax.fori_loop(0, rpt, step, 0)
        plsc.subcore_barrier()
        pltpu.sync_copy(acc_sh.at[pl.ds(base, per_tile)],
                        out.at[c, pl.ds(base, per_tile)])

    return agg_kernel


def _dinv_of(dg_ref):
    deg = dg_ref[:, 0:1] + dg_ref[:, 1:2] - 1.0
    return lax.rsqrt(deg)


def _g_body(x_ref, w_ref, dg_ref, g_ref):
    dinv = _dinv_of(dg_ref)
    h = jnp.dot(x_ref[...], w_ref[...], preferred_element_type=jnp.float32)
    g_ref[...] = h * dinv


def _fin1_body(p_ref, g_ref, dg_ref, b_ref, w_ref, x1_ref, g2_ref):
    dinv = _dinv_of(dg_ref)
    t = (p_ref[0] + p_ref[1] - g_ref[...]) * dinv + b_ref[...]
    x1 = jnp.maximum(t, 0.01 * t)
    x1_ref[...] = x1
    g2_ref[...] = jnp.dot(x1, w_ref[...],
                          preferred_element_type=jnp.float32) * dinv


def _fin2_body(p_ref, g_ref, dg_ref, b_ref, x2_ref):
    dinv = _dinv_of(dg_ref)
    t = (p_ref[0] + p_ref[1] - g_ref[...]) * dinv + b_ref[...]
    x2_ref[...] = jnp.maximum(t, 0.01 * t)


def kernel(id_embedding, edge_index, W1, b1, W2, b2):
    n, d = id_embedding.shape
    e = edge_index.shape[1]
    npad = _pad_up(n + 1, NS * 16)
    epad = _pad_up(e, LANES * NW)
    nrows = epad // LANES
    blk = 1280
    grid = npad // blk

    ei = edge_index.astype(jnp.int32)
    pad_e = jnp.full((epad - e,), n, jnp.int32)
    srcp = jnp.concatenate([ei[0], pad_e]).reshape(nrows, LANES)
    dstp = jnp.concatenate([ei[1], pad_e]).reshape(nrows, LANES)
    xpad = jnp.zeros((npad, d), jnp.float32).at[:n].set(id_embedding)

    ones2d = jnp.ones((npad, d), jnp.float32)
    degp = _make_deg(npad, nrows, d)(ones2d, srcp, dstp)
    degt = jnp.concatenate([degp[0, :, :1], degp[1, :, :1]], axis=1)

    row_spec = pl.BlockSpec((blk, d), lambda i: (i, 0))
    w_spec = pl.BlockSpec((d, d), lambda i: (0, 0))
    dg_spec = pl.BlockSpec((blk, 2), lambda i: (i, 0))
    b_spec = pl.BlockSpec((1, d), lambda i: (0, 0))
    p_spec = pl.BlockSpec((NC, blk, d), lambda i: (0, i, 0))
    row_shape = jax.ShapeDtypeStruct((npad, d), jnp.float32)

    g1 = pl.pallas_call(
        _g_body, grid=(grid,),
        in_specs=[row_spec, w_spec, dg_spec],
        out_specs=row_spec, out_shape=row_shape,
    )(xpad, W1, degt)

    agg = _make_agg(npad, nrows, d)
    p1 = agg(g1, srcp, dstp)

    x1, g2 = pl.pallas_call(
        _fin1_body, grid=(grid,),
        in_specs=[p_spec, row_spec, dg_spec, b_spec, w_spec],
        out_specs=[row_spec, row_spec], out_shape=[row_shape, row_shape],
    )(p1, g1, degt, b1.reshape(1, d), W2)

    p2 = agg(g2, srcp, dstp)

    x2 = pl.pallas_call(
        _fin2_body, grid=(grid,),
        in_specs=[p_spec, row_spec, dg_spec, b_spec],
        out_specs=row_spec, out_shape=row_shape,
    )(p2, g2, degt, b2.reshape(1, d))

    return jnp.concatenate([x1[:n], x2[:n]], axis=1)


# speedup vs baseline: 8.6509x; 1.0998x over previous
import functools

import jax
import jax.numpy as jnp
from jax import lax
from jax.experimental import pallas as pl
from jax.experimental.pallas import tpu as pltpu
from jax.experimental.pallas import tpu_sc as plsc

NC = 2
NS = 16
NW = NC * NS
LANES = 128
DEGW = 16


def _pad_up(x, m):
    return (x + m - 1) // m * m


def _make_deg(npad, nrows, d):
    rpt = nrows // NW
    per_tile = npad // NS
    mesh = plsc.VectorSubcoreMesh(core_axis_name="c", subcore_axis_name="s")

    @functools.partial(
        pl.kernel,
        mesh=mesh,
        out_type=jax.ShapeDtypeStruct((NC, npad, d), jnp.float32),
        scratch_types=[
            pltpu.VMEM((rpt, LANES), jnp.int32),
            pltpu.VMEM((rpt, LANES), jnp.int32),
            pltpu.VMEM((LANES, d), jnp.float32),
            pltpu.VMEM_SHARED((npad, d), jnp.float32),
            [pltpu.SemaphoreType.DMA] * 4,
        ],
    )
    def deg_kernel(ones2d, srcp, dstp, out, idx_sa, idx_da, ones_v, deg_sh,
                   sems):
        c = lax.axis_index("c")
        s = lax.axis_index("s")
        wid = s * NC + c
        base = s * per_tile
        pltpu.sync_copy(ones2d.at[pl.ds(base, per_tile)],
                        deg_sh.at[pl.ds(base, per_tile)])
        pltpu.sync_copy(ones2d.at[pl.ds(0, LANES)], ones_v)
        pltpu.sync_copy(srcp.at[pl.ds(wid * rpt, rpt)], idx_sa)
        pltpu.sync_copy(dstp.at[pl.ds(wid * rpt, rpt)], idx_da)
        plsc.subcore_barrier()

        def step(js, carry):
            j0 = 2 * js
            plan = [idx_sa.at[j0], idx_da.at[j0],
                    idx_sa.at[j0 + 1], idx_da.at[j0 + 1]]
            ss = [pltpu.async_copy(ones_v, deg_sh.at[si], sems[k], add=True)
                  for k, si in enumerate(plan)]
            for sd in ss:
                sd.wait()
            return carry

        lax.fori_loop(0, rpt // 2, step, 0)
        plsc.subcore_barrier()
        pltpu.sync_copy(deg_sh.at[pl.ds(base, per_tile)],
                        out.at[c, pl.ds(base, per_tile)])

    return deg_kernel


def _make_agg(npad, nrows, d):
    rpt = nrows // NW
    per_tile = npad // NS
    mesh = plsc.VectorSubcoreMesh(core_axis_name="c", subcore_axis_name="s")

    @functools.partial(
        pl.kernel,
        mesh=mesh,
        out_type=jax.ShapeDtypeStruct((NC, npad, d), jnp.float32),
        scratch_types=[
            pltpu.VMEM((rpt, LANES), jnp.int32),
            pltpu.VMEM((rpt, LANES), jnp.int32),
            pltpu.VMEM((2, LANES, d), jnp.float32),
            pltpu.VMEM_SHARED((npad, d), jnp.float32),
            [pltpu.SemaphoreType.DMA] * 4,
        ],
    )
    def agg_kernel(g, srcp, dstp, out, idx_sa, idx_da, bufs, acc_sh, sems):
        c = lax.axis_index("c")
        s = lax.axis_index("s")
        wid = s * NC + c
        base = s * per_tile
        pltpu.sync_copy(g.at[pl.ds(base, per_tile)],
                        acc_sh.at[pl.ds(base, per_tile)])
        pltpu.sync_copy(srcp.at[pl.ds(wid * rpt, rpt)], idx_sa)
        pltpu.sync_copy(dstp.at[pl.ds(wid * rpt, rpt)], idx_da)
        plsc.subcore_barrier()

        def step(js, carry):
            plan = [(idx_sa.at[js], idx_da.at[js]),
                    (idx_da.at[js], idx_sa.at[js])]
            gs = [pltpu.async_copy(g.at[gi], bufs.at[k], sems[k])
                  for k, (gi, _) in enumerate(plan)]
            ss = []
            for k, (_, si) in enumerate(plan):
                gs[k].wait()
                ss.append(pltpu.async_copy(bufs.at[k], acc_sh.at[si],
                                           sems[2 + k], add=True))
            for sd in ss:
                sd.wait()
            return carry

        lax.fori_loop(0, rpt, step, 0)
        plsc.subcore_barrier()
        pltpu.sync_copy(acc_sh.at[pl.ds(base, per_tile)],
                        out.at[c, pl.ds(base, per_tile)])

    return agg_kernel


def _dinv_of(dg_ref):
    deg = dg_ref[:, 0:1] + dg_ref[:, 1:2] - 1.0
    return lax.rsqrt(deg)


def _g_body(x_ref, w_ref, dg_ref, g_ref):
    dinv = _dinv_of(dg_ref)
    h = jnp.dot(x_ref[...], w_ref[...], preferred_element_type=jnp.float32)
    g_ref[...] = h * dinv


def _fin1_body(p_ref, g_ref, dg_ref, b_ref, w_ref, x1_ref, g2_ref):
    dinv = _dinv_of(dg_ref)
    t = (p_ref[0] + p_ref[1] - g_ref[...]) * dinv + b_ref[...]
    x1 = jnp.maximum(t, 0.01 * t)
    x1_ref[...] = x1
    g2_ref[...] = jnp.dot(x1, w_ref[...],
                          preferred_element_type=jnp.float32) * dinv


def _fin2_body(p_ref, g_ref, dg_ref, b_ref, x2_ref):
    dinv = _dinv_of(dg_ref)
    t = (p_ref[0] + p_ref[1] - g_ref[...]) * dinv + b_ref[...]
    x2_ref[...] = jnp.maximum(t, 0.01 * t)


def kernel(id_embedding, edge_index, W1, b1, W2, b2):
    n, d = id_embedding.shape
    e = edge_index.shape[1]
    npad = _pad_up(n + 1, NS * 16)
    epad = _pad_up(e, LANES * NW)
    nrows = epad // LANES
    blk = 1280
    grid = npad // blk

    ei = edge_index.astype(jnp.int32)
    pad_e = jnp.full((epad - e,), n, jnp.int32)
    srcp = jnp.concatenate([ei[0], pad_e]).reshape(nrows, LANES)
    dstp = jnp.concatenate([ei[1], pad_e]).reshape(nrows, LANES)
    xpad = jnp.zeros((npad, d), jnp.float32).at[:n].set(id_embedding)

    ones2d = jnp.ones((npad, d), jnp.float32)
    degp = _make_deg(npad, nrows, d)(ones2d, srcp, dstp)
    degt = jnp.concatenate([degp[0, :, :1], degp[1, :, :1]], axis=1)

    row_spec = pl.BlockSpec((blk, d), lambda i: (i, 0))
    w_spec = pl.BlockSpec((d, d), lambda i: (0, 0))
    dg_spec = pl.BlockSpec((blk, 2), lambda i: (i, 0))
    b_spec = pl.BlockSpec((1, d), lambda i: (0, 0))
    p_spec = pl.BlockSpec((NC, blk, d), lambda i: (0, i, 0))
    row_shape = jax.ShapeDtypeStruct((npad, d), jnp.float32)

    g1 = pl.pallas_call(
        _g_body, grid=(grid,),
        in_specs=[row_spec, w_spec, dg_spec],
        out_specs=row_spec, out_shape=row_shape,
    )(xpad, W1, degt)

    agg = _make_agg(npad, nrows, d)
    p1 = agg(g1, srcp, dstp)

    x1, g2 = pl.pallas_call(
        _fin1_body, grid=(grid,),
        in_specs=[p_spec, row_spec, dg_spec, b_spec, w_spec],
        out_specs=[row_spec, row_spec], out_shape=[row_shape, row_shape],
    )(p1, g1, degt, b1.reshape(1, d), W2)

    p2 = agg(g2, srcp, dstp)

    x2 = pl.pallas_call(
        _fin2_body, grid=(grid,),
        in_specs=[p_spec, row_spec, dg_spec, b_spec],
        out_specs=row_spec, out_shape=row_shape,
    )(p2, g2, degt, b2.reshape(1, d))

    return jnp.concatenate([x1[:n], x2[:n]], axis=1)
